# Initial kernel scaffold; baseline (speedup 1.0000x reference)
import jax
import jax.numpy as jnp
from jax.experimental import pallas as pl


def kernel(probabilities):
    raise NotImplementedError("write your pallas kernel here")


# speedup vs baseline: 1.0569x; 1.0569x over previous
import functools

import jax
import jax.numpy as jnp
from jax import lax
from jax.experimental import pallas as pl
from jax.experimental.pallas import tpu as pltpu
from jax.experimental.pallas import tpu_sc as plsc

EPS = 0.05
N = 1_000_000
NC = 2
NS = 16
NW = NC * NS
L = 16

VREGS_W = 1953
CHUNK = VREGS_W * L
MAIN = NW * CHUNK
TAIL = N - MAIN
TAIL_VREGS = TAIL // L
NACC = 7
UNROLL = 21
STEPS = VREGS_W // UNROLL
BIG = 2**31 - 1

_mesh = plsc.VectorSubcoreMesh(core_axis_name="c", subcore_axis_name="s")


@functools.partial(
    pl.kernel,
    out_type=(
        jax.ShapeDtypeStruct((NC, L), jnp.float32),
        jax.ShapeDtypeStruct((NC, L), jnp.int32),
    ),
    mesh=_mesh,
    scratch_types=[
        pltpu.VMEM((CHUNK + TAIL,), jnp.float32),
        pltpu.VMEM((L,), jnp.float32),
        pltpu.VMEM((L,), jnp.int32),
        pltpu.VMEM_SHARED((NS * L,), jnp.float32),
        pltpu.VMEM_SHARED((NS * L,), jnp.int32),
        pltpu.VMEM((NS * L,), jnp.float32),
        pltpu.VMEM((NS * L,), jnp.int32),
    ],
)
def _sc_argmax_call(p_hbm, out_val, out_idx, buf, cand_v, cand_i,
                    sh_val, sh_idx, red_v, red_i):
    c = lax.axis_index("c")
    s = lax.axis_index("s")
    wid = c * NS + s
    base = wid * CHUNK

    pltpu.sync_copy(p_hbm.at[pl.ds(base, CHUNK)], buf.at[pl.ds(0, CHUNK)])
    pltpu.sync_copy(p_hbm.at[pl.ds(MAIN, TAIL)], buf.at[pl.ds(CHUNK, TAIL)])

    iota = lax.iota(jnp.int32, L)
    neg = jnp.full((L,), -jnp.inf, jnp.float32)
    vmax = [neg for _ in range(NACC)]
    vcur = [iota + (base + k * L) for k in range(NACC)]
    vidx = [vcur[k] for k in range(NACC)]

    def body(i, carry):
        vm, vi, vc = [list(t) for t in carry]
        off0 = i * (UNROLL * L)
        for st in range(UNROLL):
            k = st % NACC
            v = buf[pl.ds(off0 + st * L, L)]
            m = v > vm[k]
            vm[k] = jnp.where(m, v, vm[k])
            vi[k] = jnp.where(m, vc[k], vi[k])
            vc[k] = vc[k] + NACC * L
        return tuple(vm), tuple(vi), tuple(vc)

    (vm, vi, _) = lax.fori_loop(
        0, STEPS, body, (tuple(vmax), tuple(vidx), tuple(vcur)))
    vm, vi = list(vm), list(vi)

    for t in range(TAIL_VREGS):
        v = buf[pl.ds(CHUNK + t * L, L)]
        tv = iota + (MAIN + t * L)
        m = v > vm[0]
        vm[0] = jnp.where(m, v, vm[0])
        vi[0] = jnp.where(m, tv, vi[0])

    bm, bi = vm[0], vi[0]
    for k in range(1, NACC):
        better = (vm[k] > bm) | ((vm[k] == bm) & (vi[k] < bi))
        bm = jnp.where(better, vm[k], bm)
        bi = jnp.where(better, vi[k], bi)

    cand_v[...] = bm
    cand_i[...] = bi
    pltpu.sync_copy(cand_v, sh_val.at[pl.ds(s * L, L)])
    pltpu.sync_copy(cand_i, sh_idx.at[pl.ds(s * L, L)])
    plsc.subcore_barrier()

    @pl.when(s == 0)
    def _():
        pltpu.sync_copy(sh_val, red_v)
        pltpu.sync_copy(sh_idx, red_i)
        fm = red_v[pl.ds(0, L)]
        fi = red_i[pl.ds(0, L)]
        for r in range(1, NS):
            rv = red_v[pl.ds(r * L, L)]
            ri = red_i[pl.ds(r * L, L)]
            better = (rv > fm) | ((rv == fm) & (ri < fi))
            fm = jnp.where(better, rv, fm)
            fi = jnp.where(better, ri, fi)
        cand_v[...] = fm
        cand_i[...] = fi
        pltpu.sync_copy(cand_v, out_val.at[c])
        pltpu.sync_copy(cand_i, out_idx.at[c])


def _sc_argmax(x):
    vals, idxs = _sc_argmax_call(x)
    vals, idxs = vals.reshape(-1), idxs.reshape(-1)
    m = jnp.max(vals)
    return jnp.min(jnp.where(vals == m, idxs, BIG)).astype(jnp.int32)


def kernel(probabilities):
    key = jax.random.key(42)
    k_branch, k_sample = jax.random.split(key)
    u = jax.random.uniform(k_branch, (), dtype=jnp.float32)

    def greedy(p):
        return _sc_argmax(p)

    def sample(p):
        g = jax.random.gumbel(k_sample, p.shape, jnp.float32)
        return _sc_argmax(jnp.log(p) + g)

    return lax.cond(u > EPS, greedy, sample, probabilities)


# speedup vs baseline: 1.3805x; 1.3062x over previous
import functools

import jax
import jax.numpy as jnp
from jax import lax
from jax.experimental import pallas as pl
from jax.experimental.pallas import tpu as pltpu
from jax.experimental.pallas import tpu_sc as plsc

EPS = 0.05
N = 1_000_000
NC = 2
NS = 16
NW = NC * NS
L = 16

VREGS_W = 1953
CHUNK = VREGS_W * L
MAIN = NW * CHUNK
TAIL = N - MAIN
TAIL_VREGS = TAIL // L
NACC = 7
UNROLL = 21
STEPS = VREGS_W // UNROLL
BIG = 2**31 - 1

_mesh = plsc.VectorSubcoreMesh(core_axis_name="c", subcore_axis_name="s")


@functools.partial(
    pl.kernel,
    out_type=(
        jax.ShapeDtypeStruct((NC, L), jnp.float32),
        jax.ShapeDtypeStruct((NC, L), jnp.int32),
    ),
    mesh=_mesh,
    scratch_types=[
        pltpu.VMEM((CHUNK + TAIL,), jnp.float32),
        pltpu.VMEM((L,), jnp.float32),
        pltpu.VMEM((L,), jnp.int32),
        pltpu.VMEM_SHARED((NS * L,), jnp.float32),
        pltpu.VMEM_SHARED((NS * L,), jnp.int32),
        pltpu.VMEM((NS * L,), jnp.float32),
        pltpu.VMEM((NS * L,), jnp.int32),
    ],
)
def _sc_argmax_call(p_hbm, out_val, out_idx, buf, cand_v, cand_i,
                    sh_val, sh_idx, red_v, red_i):
    c = lax.axis_index("c")
    s = lax.axis_index("s")
    wid = c * NS + s
    base = wid * CHUNK

    pltpu.sync_copy(p_hbm.at[pl.ds(base, CHUNK)], buf.at[pl.ds(0, CHUNK)])
    pltpu.sync_copy(p_hbm.at[pl.ds(MAIN, TAIL)], buf.at[pl.ds(CHUNK, TAIL)])

    iota = lax.iota(jnp.int32, L)
    neg = jnp.full((L,), -jnp.inf, jnp.float32)
    vmax = [neg for _ in range(NACC)]
    vcur = [iota + (base + k * L) for k in range(NACC)]
    vidx = [vcur[k] for k in range(NACC)]

    def body(i, carry):
        vm, vi, vc = [list(t) for t in carry]
        off0 = i * (UNROLL * L)
        for st in range(UNROLL):
            k = st % NACC
            v = buf[pl.ds(off0 + st * L, L)]
            m = v > vm[k]
            vm[k] = jnp.where(m, v, vm[k])
            vi[k] = jnp.where(m, vc[k], vi[k])
            vc[k] = vc[k] + NACC * L
        return tuple(vm), tuple(vi), tuple(vc)

    (vm, vi, _) = lax.fori_loop(
        0, STEPS, body, (tuple(vmax), tuple(vidx), tuple(vcur)))
    vm, vi = list(vm), list(vi)

    for t in range(TAIL_VREGS):
        v = buf[pl.ds(CHUNK + t * L, L)]
        tv = iota + (MAIN + t * L)
        m = v > vm[0]
        vm[0] = jnp.where(m, v, vm[0])
        vi[0] = jnp.where(m, tv, vi[0])

    bm, bi = vm[0], vi[0]
    for k in range(1, NACC):
        better = (vm[k] > bm) | ((vm[k] == bm) & (vi[k] < bi))
        bm = jnp.where(better, vm[k], bm)
        bi = jnp.where(better, vi[k], bi)

    cand_v[...] = bm
    cand_i[...] = bi
    pltpu.sync_copy(cand_v, sh_val.at[pl.ds(s * L, L)])
    pltpu.sync_copy(cand_i, sh_idx.at[pl.ds(s * L, L)])
    plsc.subcore_barrier()

    @pl.when(s == 0)
    def _():
        pltpu.sync_copy(sh_val, red_v)
        pltpu.sync_copy(sh_idx, red_i)
        fm = red_v[pl.ds(0, L)]
        fi = red_i[pl.ds(0, L)]
        for r in range(1, NS):
            rv = red_v[pl.ds(r * L, L)]
            ri = red_i[pl.ds(r * L, L)]
            better = (rv > fm) | ((rv == fm) & (ri < fi))
            fm = jnp.where(better, rv, fm)
            fi = jnp.where(better, ri, fi)
        cand_v[...] = fm
        cand_i[...] = fi
        pltpu.sync_copy(cand_v, out_val.at[c])
        pltpu.sync_copy(cand_i, out_idx.at[c])


def _sc_argmax(x):
    vals, idxs = _sc_argmax_call(x)
    vals, idxs = vals.reshape(-1), idxs.reshape(-1)
    m = jnp.max(vals)
    return jnp.min(jnp.where(vals == m, idxs, BIG)).astype(jnp.int32)


_KEY = jax.random.key(42)
_K_BRANCH, _K_SAMPLE = jax.random.split(_KEY)
_U = float(jax.random.uniform(_K_BRANCH, (), dtype=jnp.float32))


def kernel(probabilities):
    if _U > EPS:
        return _sc_argmax(probabilities)
    g = jax.random.gumbel(_K_SAMPLE, probabilities.shape, jnp.float32)
    return _sc_argmax(jnp.log(probabilities) + g)


# speedup vs baseline: 1.3806x; 1.0001x over previous
import functools

import jax
import jax.numpy as jnp
from jax import lax
from jax.experimental import pallas as pl
from jax.experimental.pallas import tpu as pltpu
from jax.experimental.pallas import tpu_sc as plsc

EPS = 0.05
N = 1_000_000
NC = 2
NS = 16
NW = NC * NS
L = 16

VREGS_W = 1953
CHUNK = VREGS_W * L
MAIN = NW * CHUNK
TAIL = N - MAIN
TAIL_VREGS = TAIL // L
NACC = 7
UNROLL = 21
STEPS = VREGS_W // UNROLL
BIG = 2**31 - 1

_mesh = plsc.VectorSubcoreMesh(core_axis_name="c", subcore_axis_name="s")


@functools.partial(
    pl.kernel,
    out_type=(
        jax.ShapeDtypeStruct((NC, L), jnp.float32),
        jax.ShapeDtypeStruct((NC, L), jnp.int32),
    ),
    mesh=_mesh,
    scratch_types=[
        pltpu.VMEM((CHUNK + TAIL,), jnp.float32),
        pltpu.VMEM((L,), jnp.float32),
        pltpu.VMEM((L,), jnp.int32),
        pltpu.VMEM_SHARED((NS * L,), jnp.float32),
        pltpu.VMEM_SHARED((NS * L,), jnp.int32),
        pltpu.VMEM((NS * L,), jnp.float32),
        pltpu.VMEM((NS * L,), jnp.int32),
    ],
)
def _sc_argmax_call(p_hbm, out_val, out_idx, buf, cand_v, cand_i,
                    sh_val, sh_idx, red_v, red_i):
    c = lax.axis_index("c")
    s = lax.axis_index("s")
    wid = c * NS + s
    base = wid * CHUNK

    pltpu.sync_copy(p_hbm.at[pl.ds(base, CHUNK)], buf.at[pl.ds(0, CHUNK)])
    pltpu.sync_copy(p_hbm.at[pl.ds(MAIN, TAIL)], buf.at[pl.ds(CHUNK, TAIL)])

    iota = lax.iota(jnp.int32, L)
    neg = jnp.full((L,), -jnp.inf, jnp.float32)
    vmax = [neg for _ in range(NACC)]
    vcur = [iota + (base + k * L) for k in range(NACC)]
    vidx = [vcur[k] for k in range(NACC)]

    def body(i, carry):
        vm, vi, vc = [list(t) for t in carry]
        off0 = i * (UNROLL * L)
        for st in range(UNROLL):
            k = st % NACC
            v = buf[pl.ds(off0 + st * L, L)]
            m = v > vm[k]
            vm[k] = jnp.where(m, v, vm[k])
            vi[k] = jnp.where(m, vc[k], vi[k])
            vc[k] = vc[k] + NACC * L
        return tuple(vm), tuple(vi), tuple(vc)

    (vm, vi, _) = lax.fori_loop(
        0, STEPS, body, (tuple(vmax), tuple(vidx), tuple(vcur)))
    vm, vi = list(vm), list(vi)

    for t in range(TAIL_VREGS):
        v = buf[pl.ds(CHUNK + t * L, L)]
        tv = iota + (MAIN + t * L)
        m = v > vm[0]
        vm[0] = jnp.where(m, v, vm[0])
        vi[0] = jnp.where(m, tv, vi[0])

    bm, bi = vm[0], vi[0]
    for k in range(1, NACC):
        better = (vm[k] > bm) | ((vm[k] == bm) & (vi[k] < bi))
        bm = jnp.where(better, vm[k], bm)
        bi = jnp.where(better, vi[k], bi)

    cand_v[...] = bm
    cand_i[...] = bi
    pltpu.sync_copy(cand_v, sh_val.at[pl.ds(s * L, L)])
    pltpu.sync_copy(cand_i, sh_idx.at[pl.ds(s * L, L)])
    plsc.subcore_barrier()

    @pl.when(s == 0)
    def _():
        pltpu.sync_copy(sh_val, red_v)
        pltpu.sync_copy(sh_idx, red_i)
        fm = red_v[pl.ds(0, L)]
        fi = red_i[pl.ds(0, L)]
        for r in range(1, NS):
            rv = red_v[pl.ds(r * L, L)]
            ri = red_i[pl.ds(r * L, L)]
            better = (rv > fm) | ((rv == fm) & (ri < fi))
            fm = jnp.where(better, rv, fm)
            fi = jnp.where(better, ri, fi)
        cand_v[...] = fm
        cand_i[...] = fi
        pltpu.sync_copy(cand_v, out_val.at[c])
        pltpu.sync_copy(cand_i, out_idx.at[c])


def _sc_argmax(x):
    vals, idxs = _sc_argmax_call(x)
    vals, idxs = vals.reshape(-1), idxs.reshape(-1)
    m = jnp.max(vals)
    return jnp.min(jnp.where(vals == m, idxs, BIG)).astype(jnp.int32)


_U = 0.5302608013153076


def kernel(probabilities):
    if _U > EPS:
        return _sc_argmax(probabilities)
    k_sample = jax.random.split(jax.random.key(42))[1]
    g = jax.random.gumbel(k_sample, probabilities.shape, jnp.float32)
    return _sc_argmax(jnp.log(probabilities) + g)
